# baseline (speedup 1.0000x reference)
import functools

import jax
import jax.numpy as jnp
from jax.experimental import pallas as pl
from jax.experimental.pallas import tpu as pltpu

_VMEM_LIMIT = 48 * 1024 * 1024


def _conv_kernel(x_ref, w_ref, scale_ref, shift_ref, y_ref, stats_ref, *,
                 cin, cout, apply_bn_relu):
    B, H, W, _ = x_ref.shape
    x = x_ref[...]
    if apply_bn_relu:
        x = jnp.maximum(x.astype(jnp.float32) * scale_ref[...] + shift_ref[...],
                        0.0).astype(jnp.bfloat16)

    zcol = jnp.zeros((B, H, 1, cin), jnp.bfloat16)
    x_l = jnp.concatenate([zcol, x[:, :, :-1, :]], axis=2)
    x_r = jnp.concatenate([x[:, :, 1:, :], zcol], axis=2)

    m = B * H * W
    z = jnp.dot(x_l.reshape(m, cin), w_ref[0],
                preferred_element_type=jnp.float32)
    z = z + jnp.dot(x.reshape(m, cin), w_ref[1],
                    preferred_element_type=jnp.float32)
    z = z + jnp.dot(x_r.reshape(m, cin), w_ref[2],
                    preferred_element_type=jnp.float32)
    z = z.reshape(B, H, W, 3 * cout)

    z0 = z[..., :cout]
    z1 = z[..., cout:2 * cout]
    z2 = z[..., 2 * cout:]
    zrow = jnp.zeros((B, 1, W, cout), jnp.float32)
    y = (z1 + jnp.concatenate([zrow, z0[:, :-1]], axis=1)
            + jnp.concatenate([z2[:, 1:], zrow], axis=1))

    y_ref[...] = y.astype(y_ref.dtype)

    yf = y.reshape(m, cout)
    gram = jax.lax.dot_general(yf, yf, (((0,), (0,)), ((), ())),
                               preferred_element_type=jnp.float32)
    cs = jnp.dot(jnp.ones((1, m), jnp.float32), yf,
                 preferred_element_type=jnp.float32)
    stats_ref[...] = jnp.concatenate([gram, cs], axis=0)[None]


def _conv(x, wmat, scale, shift, *, apply_bn_relu, out_dtype, block_n):
    N, H, W, cin = x.shape
    cout = wmat.shape[-1] // 3
    grid = N // block_n
    body = functools.partial(_conv_kernel, cin=cin, cout=cout,
                             apply_bn_relu=apply_bn_relu)
    return pl.pallas_call(
        body,
        out_shape=(jax.ShapeDtypeStruct((N, H, W, cout), out_dtype),
                   jax.ShapeDtypeStruct((grid, cout + 1, cout), jnp.float32)),
        grid=(grid,),
        in_specs=[
            pl.BlockSpec((block_n, H, W, cin), lambda n: (n, 0, 0, 0)),
            pl.BlockSpec(wmat.shape, lambda n: (0, 0, 0)),
            pl.BlockSpec((1, cin), lambda n: (0, 0)),
            pl.BlockSpec((1, cin), lambda n: (0, 0)),
        ],
        out_specs=(pl.BlockSpec((block_n, H, W, cout), lambda n: (n, 0, 0, 0)),
                   pl.BlockSpec((1, cout + 1, cout), lambda n: (n, 0, 0))),
        compiler_params=pltpu.CompilerParams(
            dimension_semantics=("parallel",),
            vmem_limit_bytes=_VMEM_LIMIT),
    )(x, wmat, scale, shift)


def _bn_relu_t_kernel(x_ref, scale_ref, shift_ref, o_ref):
    a = jnp.maximum(
        x_ref[...].astype(jnp.float32) * scale_ref[...] + shift_ref[...], 0.0)
    o_ref[...] = jnp.swapaxes(a, 1, 2)


def _bn_relu_t(y, scale, shift, *, block_n):
    N, M, c = y.shape
    grid = N // block_n
    return pl.pallas_call(
        _bn_relu_t_kernel,
        out_shape=jax.ShapeDtypeStruct((N, c, M), jnp.float32),
        grid=(grid,),
        in_specs=[
            pl.BlockSpec((block_n, M, c), lambda n: (n, 0, 0)),
            pl.BlockSpec((1, c), lambda n: (0, 0)),
            pl.BlockSpec((1, c), lambda n: (0, 0)),
        ],
        out_specs=pl.BlockSpec((block_n, c, M), lambda n: (n, 0, 0)),
        compiler_params=pltpu.CompilerParams(
            dimension_semantics=("parallel",),
            vmem_limit_bytes=_VMEM_LIMIT),
    )(y, scale, shift)


def kernel(x_nchw, w1, g1, b1, w2, g2, b2, eps=1e-5):
    N, cin, H, W = x_nchw.shape
    c1 = w1.shape[-1]
    c2 = w2.shape[-1]
    M = H * W
    count = N * M
    block_n = 8

    x = jnp.transpose(x_nchw, (0, 2, 3, 1)).astype(jnp.bfloat16)

    wm1 = jnp.transpose(w1, (1, 2, 0, 3)).reshape(3, cin, 3 * c1)
    wm2 = jnp.transpose(w2, (1, 2, 0, 3)).reshape(3, c1, 3 * c2)
    wm1 = wm1.astype(jnp.bfloat16)
    wm2 = wm2.astype(jnp.bfloat16)

    one = jnp.ones((1, cin), jnp.float32)
    zero = jnp.zeros((1, cin), jnp.float32)

    y1, st1 = _conv(x, wm1, one, zero, apply_bn_relu=False,
                    out_dtype=jnp.bfloat16, block_n=block_n)
    s1 = jnp.sum(st1, axis=0)
    mu1 = s1[c1] / count
    var1 = jnp.diagonal(s1[:c1]) / count - mu1 * mu1
    scale1 = g1 / jnp.sqrt(var1 + eps)
    shift1 = b1 - mu1 * scale1

    y2, st2 = _conv(y1, wm2, scale1[None], shift1[None],
                    apply_bn_relu=True, out_dtype=jnp.bfloat16, block_n=block_n)
    s2 = jnp.sum(st2, axis=0)
    mu2 = s2[c2] / count
    var2 = jnp.diagonal(s2[:c2]) / count - mu2 * mu2
    scale2 = g2 / jnp.sqrt(var2 + eps)
    shift2 = b2 - mu2 * scale2

    a2 = _bn_relu_t(y2.reshape(N, M, c2), scale2[None], shift2[None],
                    block_n=block_n)

    return a2.reshape(N, c2, H, W)


# speedup vs baseline: 1.3475x; 1.3475x over previous
import functools

import jax
import jax.numpy as jnp
from jax.experimental import pallas as pl
from jax.experimental.pallas import tpu as pltpu

_VMEM_LIMIT = 48 * 1024 * 1024


def _conv_kernel(x_ref, w_ref, scale_ref, shift_ref, y_ref, stats_ref, *,
                 cin, cout, apply_bn_relu):
    B, H, W, _ = x_ref.shape
    x = x_ref[...]
    if apply_bn_relu:
        x = jnp.maximum(x.astype(jnp.float32) * scale_ref[...] + shift_ref[...],
                        0.0).astype(jnp.bfloat16)

    zcol = jnp.zeros((B, H, 1, cin), jnp.bfloat16)
    x_l = jnp.concatenate([zcol, x[:, :, :-1, :]], axis=2)
    x_r = jnp.concatenate([x[:, :, 1:, :], zcol], axis=2)

    m = B * H * W
    z = jnp.dot(x_l.reshape(m, cin), w_ref[0],
                preferred_element_type=jnp.float32)
    z = z + jnp.dot(x.reshape(m, cin), w_ref[1],
                    preferred_element_type=jnp.float32)
    z = z + jnp.dot(x_r.reshape(m, cin), w_ref[2],
                    preferred_element_type=jnp.float32)
    z = z.reshape(B, H, W, 3 * cout)

    z0 = z[..., :cout]
    z1 = z[..., cout:2 * cout]
    z2 = z[..., 2 * cout:]
    zrow = jnp.zeros((B, 1, W, cout), jnp.float32)
    y = (z1 + jnp.concatenate([zrow, z0[:, :-1]], axis=1)
            + jnp.concatenate([z2[:, 1:], zrow], axis=1))

    y_ref[...] = y.astype(y_ref.dtype)

    yf = y.reshape(m, cout)
    cs = jnp.sum(yf, axis=0, keepdims=True)
    css = jnp.sum(yf * yf, axis=0, keepdims=True)
    stats_ref[...] = jnp.concatenate([cs, css], axis=0)[None]


def _conv(x, wmat, scale, shift, *, apply_bn_relu, out_dtype, block_n):
    N, H, W, cin = x.shape
    cout = wmat.shape[-1] // 3
    grid = N // block_n
    body = functools.partial(_conv_kernel, cin=cin, cout=cout,
                             apply_bn_relu=apply_bn_relu)
    return pl.pallas_call(
        body,
        out_shape=(jax.ShapeDtypeStruct((N, H, W, cout), out_dtype),
                   jax.ShapeDtypeStruct((grid, 2, cout), jnp.float32)),
        grid=(grid,),
        in_specs=[
            pl.BlockSpec((block_n, H, W, cin), lambda n: (n, 0, 0, 0)),
            pl.BlockSpec(wmat.shape, lambda n: (0, 0, 0)),
            pl.BlockSpec((1, cin), lambda n: (0, 0)),
            pl.BlockSpec((1, cin), lambda n: (0, 0)),
        ],
        out_specs=(pl.BlockSpec((block_n, H, W, cout), lambda n: (n, 0, 0, 0)),
                   pl.BlockSpec((1, 2, cout), lambda n: (n, 0, 0))),
        compiler_params=pltpu.CompilerParams(
            dimension_semantics=("parallel",),
            vmem_limit_bytes=_VMEM_LIMIT),
    )(x, wmat, scale, shift)


def _bn_relu_t_kernel(x_ref, scale_ref, shift_ref, o_ref):
    a = jnp.maximum(
        x_ref[...].astype(jnp.float32) * scale_ref[...] + shift_ref[...], 0.0)
    o_ref[...] = jnp.swapaxes(a, 1, 2)


def _bn_relu_t(y, scale, shift, *, block_n):
    N, M, c = y.shape
    grid = N // block_n
    return pl.pallas_call(
        _bn_relu_t_kernel,
        out_shape=jax.ShapeDtypeStruct((N, c, M), jnp.float32),
        grid=(grid,),
        in_specs=[
            pl.BlockSpec((block_n, M, c), lambda n: (n, 0, 0)),
            pl.BlockSpec((1, c), lambda n: (0, 0)),
            pl.BlockSpec((1, c), lambda n: (0, 0)),
        ],
        out_specs=pl.BlockSpec((block_n, c, M), lambda n: (n, 0, 0)),
        compiler_params=pltpu.CompilerParams(
            dimension_semantics=("parallel",),
            vmem_limit_bytes=_VMEM_LIMIT),
    )(y, scale, shift)


def kernel(x_nchw, w1, g1, b1, w2, g2, b2, eps=1e-5):
    N, cin, H, W = x_nchw.shape
    c1 = w1.shape[-1]
    c2 = w2.shape[-1]
    M = H * W
    count = N * M
    block_n = 16

    x = jnp.transpose(x_nchw, (0, 2, 3, 1)).astype(jnp.bfloat16)

    wm1 = jnp.transpose(w1, (1, 2, 0, 3)).reshape(3, cin, 3 * c1)
    wm2 = jnp.transpose(w2, (1, 2, 0, 3)).reshape(3, c1, 3 * c2)
    wm1 = wm1.astype(jnp.bfloat16)
    wm2 = wm2.astype(jnp.bfloat16)

    one = jnp.ones((1, cin), jnp.float32)
    zero = jnp.zeros((1, cin), jnp.float32)

    y1, st1 = _conv(x, wm1, one, zero, apply_bn_relu=False,
                    out_dtype=jnp.bfloat16, block_n=block_n)
    s1 = jnp.sum(st1, axis=0)
    mu1 = s1[0] / count
    var1 = s1[1] / count - mu1 * mu1
    scale1 = g1 / jnp.sqrt(var1 + eps)
    shift1 = b1 - mu1 * scale1

    y2, st2 = _conv(y1, wm2, scale1[None], shift1[None],
                    apply_bn_relu=True, out_dtype=jnp.bfloat16, block_n=block_n)
    s2 = jnp.sum(st2, axis=0)
    mu2 = s2[0] / count
    var2 = s2[1] / count - mu2 * mu2
    scale2 = g2 / jnp.sqrt(var2 + eps)
    shift2 = b2 - mu2 * scale2

    a2 = _bn_relu_t(y2.reshape(N, M, c2), scale2[None], shift2[None],
                    block_n=block_n)

    return a2.reshape(N, c2, H, W)


# speedup vs baseline: 2.7004x; 2.0040x over previous
import functools

import jax
import jax.numpy as jnp
from jax.experimental import pallas as pl
from jax.experimental.pallas import tpu as pltpu

_VMEM_LIMIT = 48 * 1024 * 1024


def _conv_kernel(x_ref, w_ref, maskl_ref, maskr_ref, scale_ref, shift_ref,
                 y_ref, stats_ref, *, cin, cout, width, apply_bn_relu):
    B, _, M = x_ref.shape
    x = x_ref[...]
    if apply_bn_relu:
        x = jnp.maximum(x.astype(jnp.float32) * scale_ref[...] + shift_ref[...],
                        0.0).astype(jnp.bfloat16)
    else:
        x = x.astype(jnp.bfloat16)

    xa = jnp.concatenate([x[b] for b in range(B)], axis=1)

    zc = jnp.zeros((cin, 1), jnp.bfloat16)
    x_l = jnp.concatenate([zc, xa[:, :-1]], axis=1) * maskl_ref[...]
    x_r = jnp.concatenate([xa[:, 1:], zc], axis=1) * maskr_ref[...]

    z = jnp.dot(w_ref[0], x_l, preferred_element_type=jnp.float32)
    z = z + jnp.dot(w_ref[1], xa, preferred_element_type=jnp.float32)
    z = z + jnp.dot(w_ref[2], x_r, preferred_element_type=jnp.float32)

    acc = jnp.zeros((cout, M), jnp.float32)
    acc2 = jnp.zeros((cout, M), jnp.float32)
    zrow = jnp.zeros((cout, width), jnp.float32)
    for b in range(B):
        lo = b * M
        z0 = z[:cout, lo:lo + M]
        z1 = z[cout:2 * cout, lo:lo + M]
        z2 = z[2 * cout:, lo:lo + M]
        y_b = (z1 + jnp.concatenate([zrow, z0[:, :-width]], axis=1)
                  + jnp.concatenate([z2[:, width:], zrow], axis=1))
        y_ref[b] = y_b.astype(y_ref.dtype)
        acc = acc + y_b
        acc2 = acc2 + y_b * y_b

    cs = jnp.sum(acc, axis=1, keepdims=True)
    css = jnp.sum(acc2, axis=1, keepdims=True)
    stats_ref[...] = jnp.stack([cs, css], axis=0)[None]


def _conv(x, wmat, maskl, maskr, scale, shift, *, width, apply_bn_relu,
          out_dtype, block_n):
    N, cin, M = x.shape
    cout = wmat.shape[1] // 3
    grid = N // block_n
    body = functools.partial(_conv_kernel, cin=cin, cout=cout, width=width,
                             apply_bn_relu=apply_bn_relu)
    return pl.pallas_call(
        body,
        out_shape=(jax.ShapeDtypeStruct((N, cout, M), out_dtype),
                   jax.ShapeDtypeStruct((grid, 2, cout, 1), jnp.float32)),
        grid=(grid,),
        in_specs=[
            pl.BlockSpec((block_n, cin, M), lambda n: (n, 0, 0)),
            pl.BlockSpec(wmat.shape, lambda n: (0, 0, 0)),
            pl.BlockSpec(maskl.shape, lambda n: (0, 0)),
            pl.BlockSpec(maskr.shape, lambda n: (0, 0)),
            pl.BlockSpec(scale.shape, lambda n: (0, 0)),
            pl.BlockSpec(shift.shape, lambda n: (0, 0)),
        ],
        out_specs=(pl.BlockSpec((block_n, cout, M), lambda n: (n, 0, 0)),
                   pl.BlockSpec((1, 2, cout, 1), lambda n: (n, 0, 0, 0))),
        compiler_params=pltpu.CompilerParams(
            dimension_semantics=("parallel",),
            vmem_limit_bytes=_VMEM_LIMIT),
    )(x, wmat, maskl, maskr, scale, shift)


def _bn_relu_kernel(x_ref, scale_ref, shift_ref, o_ref):
    o_ref[...] = jnp.maximum(
        x_ref[...].astype(jnp.float32) * scale_ref[...] + shift_ref[...], 0.0)


def _bn_relu(y, scale, shift, *, block_n):
    N, c, M = y.shape
    grid = N // block_n
    return pl.pallas_call(
        _bn_relu_kernel,
        out_shape=jax.ShapeDtypeStruct((N, c, M), jnp.float32),
        grid=(grid,),
        in_specs=[
            pl.BlockSpec((block_n, c, M), lambda n: (n, 0, 0)),
            pl.BlockSpec(scale.shape, lambda n: (0, 0)),
            pl.BlockSpec(shift.shape, lambda n: (0, 0)),
        ],
        out_specs=pl.BlockSpec((block_n, c, M), lambda n: (n, 0, 0)),
        compiler_params=pltpu.CompilerParams(
            dimension_semantics=("parallel",),
            vmem_limit_bytes=_VMEM_LIMIT),
    )(y, scale, shift)


def kernel(x_nchw, w1, g1, b1, w2, g2, b2, eps=1e-5):
    N, cin, H, W = x_nchw.shape
    c1 = w1.shape[-1]
    c2 = w2.shape[-1]
    M = H * W
    count = N * M
    block_n = 16

    x = x_nchw.reshape(N, cin, M)

    wm1 = jnp.transpose(w1, (1, 0, 3, 2)).reshape(3, 3 * c1, cin)
    wm2 = jnp.transpose(w2, (1, 0, 3, 2)).reshape(3, 3 * c2, c1)
    wm1 = wm1.astype(jnp.bfloat16)
    wm2 = wm2.astype(jnp.bfloat16)

    col = jnp.arange(block_n * M, dtype=jnp.int32) % W
    maskl = (col != 0).astype(jnp.bfloat16)[None]
    maskr = (col != W - 1).astype(jnp.bfloat16)[None]

    one = jnp.ones((cin, 1), jnp.float32)
    zero = jnp.zeros((cin, 1), jnp.float32)

    y1, st1 = _conv(x, wm1, maskl, maskr, one, zero, width=W,
                    apply_bn_relu=False, out_dtype=jnp.bfloat16,
                    block_n=block_n)
    s1 = jnp.sum(st1, axis=0)[..., 0]
    mu1 = s1[0] / count
    var1 = s1[1] / count - mu1 * mu1
    scale1 = g1 / jnp.sqrt(var1 + eps)
    shift1 = b1 - mu1 * scale1

    y2, st2 = _conv(y1, wm2, maskl, maskr, scale1[:, None], shift1[:, None],
                    width=W, apply_bn_relu=True, out_dtype=jnp.bfloat16,
                    block_n=block_n)
    s2 = jnp.sum(st2, axis=0)[..., 0]
    mu2 = s2[0] / count
    var2 = s2[1] / count - mu2 * mu2
    scale2 = g2 / jnp.sqrt(var2 + eps)
    shift2 = b2 - mu2 * scale2

    a2 = _bn_relu(y2, scale2[:, None], shift2[:, None], block_n=block_n)

    return a2.reshape(N, c2, H, W)


# speedup vs baseline: 3.2711x; 1.2113x over previous
import functools

import jax
import jax.numpy as jnp
from jax.experimental import pallas as pl
from jax.experimental.pallas import tpu as pltpu

_VMEM_LIMIT = 48 * 1024 * 1024


def _conv_kernel(x_ref, w_ref, maskl_ref, maskr_ref, scale_ref, shift_ref,
                 y_ref, stats_ref, *, cin, cout, width, apply_bn_relu):
    B, _, M = x_ref.shape
    x = x_ref[...]
    if apply_bn_relu:
        x = jnp.maximum(x.astype(jnp.float32) * scale_ref[...] + shift_ref[...],
                        0.0).astype(jnp.bfloat16)
    else:
        x = x.astype(jnp.bfloat16)

    xa = jnp.concatenate([x[b] for b in range(B)], axis=1)

    zc = jnp.zeros((cin, 1), jnp.bfloat16)
    x_l = jnp.concatenate([zc, xa[:, :-1]], axis=1) * maskl_ref[...]
    x_r = jnp.concatenate([xa[:, 1:], zc], axis=1) * maskr_ref[...]

    x3 = jnp.concatenate([x_l, xa, x_r], axis=0)
    z = jnp.dot(w_ref[...], x3, preferred_element_type=jnp.float32)

    acc = jnp.zeros((cout, M), jnp.float32)
    acc2 = jnp.zeros((cout, M), jnp.float32)
    zrow = jnp.zeros((cout, width), jnp.float32)
    for b in range(B):
        lo = b * M
        z0 = z[:cout, lo:lo + M]
        z1 = z[cout:2 * cout, lo:lo + M]
        z2 = z[2 * cout:, lo:lo + M]
        y_b = (z1 + jnp.concatenate([zrow, z0[:, :-width]], axis=1)
                  + jnp.concatenate([z2[:, width:], zrow], axis=1))
        y_ref[b] = y_b.astype(y_ref.dtype)
        acc = acc + y_b
        acc2 = acc2 + y_b * y_b

    cs = jnp.sum(acc, axis=1, keepdims=True)
    css = jnp.sum(acc2, axis=1, keepdims=True)
    stats_ref[...] = jnp.stack([cs, css], axis=0)[None]


def _conv(x, wmat, maskl, maskr, scale, shift, *, width, apply_bn_relu,
          out_dtype, block_n):
    N, cin, M = x.shape
    cout = wmat.shape[0] // 3
    grid = N // block_n
    body = functools.partial(_conv_kernel, cin=cin, cout=cout, width=width,
                             apply_bn_relu=apply_bn_relu)
    return pl.pallas_call(
        body,
        out_shape=(jax.ShapeDtypeStruct((N, cout, M), out_dtype),
                   jax.ShapeDtypeStruct((grid, 2, cout, 1), jnp.float32)),
        grid=(grid,),
        in_specs=[
            pl.BlockSpec((block_n, cin, M), lambda n: (n, 0, 0)),
            pl.BlockSpec(wmat.shape, lambda n: (0, 0)),
            pl.BlockSpec(maskl.shape, lambda n: (0, 0)),
            pl.BlockSpec(maskr.shape, lambda n: (0, 0)),
            pl.BlockSpec(scale.shape, lambda n: (0, 0)),
            pl.BlockSpec(shift.shape, lambda n: (0, 0)),
        ],
        out_specs=(pl.BlockSpec((block_n, cout, M), lambda n: (n, 0, 0)),
                   pl.BlockSpec((1, 2, cout, 1), lambda n: (n, 0, 0, 0))),
        compiler_params=pltpu.CompilerParams(
            dimension_semantics=("parallel",),
            vmem_limit_bytes=_VMEM_LIMIT),
    )(x, wmat, maskl, maskr, scale, shift)


def _bn_relu_kernel(x_ref, scale_ref, shift_ref, o_ref):
    o_ref[...] = jnp.maximum(
        x_ref[...].astype(jnp.float32) * scale_ref[...] + shift_ref[...], 0.0)


def _bn_relu(y, scale, shift, *, block_n):
    N, c, M = y.shape
    grid = N // block_n
    return pl.pallas_call(
        _bn_relu_kernel,
        out_shape=jax.ShapeDtypeStruct((N, c, M), jnp.float32),
        grid=(grid,),
        in_specs=[
            pl.BlockSpec((block_n, c, M), lambda n: (n, 0, 0)),
            pl.BlockSpec(scale.shape, lambda n: (0, 0)),
            pl.BlockSpec(shift.shape, lambda n: (0, 0)),
        ],
        out_specs=pl.BlockSpec((block_n, c, M), lambda n: (n, 0, 0)),
        compiler_params=pltpu.CompilerParams(
            dimension_semantics=("parallel",),
            vmem_limit_bytes=_VMEM_LIMIT),
    )(y, scale, shift)


def kernel(x_nchw, w1, g1, b1, w2, g2, b2, eps=1e-5):
    N, cin, H, W = x_nchw.shape
    c1 = w1.shape[-1]
    c2 = w2.shape[-1]
    M = H * W
    count = N * M
    block_n = 16

    x = x_nchw.reshape(N, cin, M)

    wm1 = jnp.transpose(w1, (0, 3, 1, 2)).reshape(3 * c1, 3 * cin)
    wm2 = jnp.transpose(w2, (0, 3, 1, 2)).reshape(3 * c2, 3 * c1)
    wm1 = wm1.astype(jnp.bfloat16)
    wm2 = wm2.astype(jnp.bfloat16)

    col = jnp.arange(block_n * M, dtype=jnp.int32) % W
    maskl = (col != 0).astype(jnp.bfloat16)[None]
    maskr = (col != W - 1).astype(jnp.bfloat16)[None]

    one = jnp.ones((cin, 1), jnp.float32)
    zero = jnp.zeros((cin, 1), jnp.float32)

    y1, st1 = _conv(x, wm1, maskl, maskr, one, zero, width=W,
                    apply_bn_relu=False, out_dtype=jnp.bfloat16,
                    block_n=block_n)
    s1 = jnp.sum(st1, axis=0)[..., 0]
    mu1 = s1[0] / count
    var1 = s1[1] / count - mu1 * mu1
    scale1 = g1 / jnp.sqrt(var1 + eps)
    shift1 = b1 - mu1 * scale1

    y2, st2 = _conv(y1, wm2, maskl, maskr, scale1[:, None], shift1[:, None],
                    width=W, apply_bn_relu=True, out_dtype=jnp.bfloat16,
                    block_n=block_n)
    s2 = jnp.sum(st2, axis=0)[..., 0]
    mu2 = s2[0] / count
    var2 = s2[1] / count - mu2 * mu2
    scale2 = g2 / jnp.sqrt(var2 + eps)
    shift2 = b2 - mu2 * scale2

    a2 = _bn_relu(y2, scale2[:, None], shift2[:, None], block_n=block_n)

    return a2.reshape(N, c2, H, W)
